# baseline (speedup 1.0000x reference)
import functools
import math

import jax
import jax.numpy as jnp
from jax import lax
from jax.experimental import pallas as pl
from jax.experimental.pallas import tpu as pltpu
from jax.experimental.pallas import tpu_sc as plsc

_NC = 2
_NS = 16
_NW = _NC * _NS
_LANES = 16
_CHUNK_ROWS = 8
_NBUF = 12


def _sc_body(scale, n_chunks, d, in_hbm, out_hbm, *scratch):
    bufs = scratch[:_NBUF]
    isems = scratch[_NBUF:2 * _NBUF]
    osems = scratch[2 * _NBUF:3 * _NBUF]
    wid = lax.axis_index("s") * _NC + lax.axis_index("c")
    base = wid * (n_chunks * _CHUNK_ROWS)
    vecs_per_row = d // _LANES
    assert vecs_per_row & (vecs_per_row - 1) == 0
    row_shift = vecs_per_row.bit_length() - 1
    vecs = _CHUNK_ROWS * vecs_per_row

    def src(c):
        return in_hbm.at[pl.ds(base + c * _CHUNK_ROWS, _CHUNK_ROWS), :]

    def dst(c):
        return out_hbm.at[pl.ds(base + c * _CHUNK_ROWS, _CHUNK_ROWS), :]

    in_h = {}
    out_h = {}
    for j in range(min(_NBUF - 1, n_chunks)):
        in_h[j] = pltpu.async_copy(src(j), bufs[j % _NBUF], isems[j % _NBUF])
    for c in range(n_chunks):
        b = c % _NBUF
        nxt = c + _NBUF - 1
        if nxt < n_chunks:
            if c - 1 >= 0:
                out_h[c - 1].wait()
            in_h[nxt] = pltpu.async_copy(
                src(nxt), bufs[nxt % _NBUF], isems[nxt % _NBUF])
        in_h[c].wait()

        @plsc.parallel_loop(0, vecs, step=1, unroll=8)
        def _scale_one(i):
            r = lax.shift_right_logical(i, row_shift)
            col = pl.multiple_of(
                lax.shift_left(lax.bitwise_and(i, vecs_per_row - 1), 4),
                _LANES)
            sl = pl.ds(col, _LANES)
            bufs[b][r, sl] = bufs[b][r, sl] * scale

        out_h[c] = pltpu.async_copy(bufs[b], dst(c), osems[b])
    for c in range(max(0, n_chunks - _NBUF), n_chunks):
        out_h[c].wait()


def kernel(x, embed):
    s = x.shape[-2]
    d = embed.shape[-1]
    scale = 1.0 / math.sqrt(d)
    assert s % (_NW * _CHUNK_ROWS) == 0 and d % _LANES == 0
    n_chunks = s // (_NW * _CHUNK_ROWS)

    mesh = plsc.VectorSubcoreMesh(
        core_axis_name="c", subcore_axis_name="s",
        num_cores=_NC, num_subcores=_NS)
    run = pl.kernel(
        functools.partial(_sc_body, scale, n_chunks, d),
        out_type=jax.ShapeDtypeStruct((s, d), embed.dtype),
        mesh=mesh,
        scratch_types=(
            [pltpu.VMEM((_CHUNK_ROWS, d), embed.dtype) for _ in range(_NBUF)]
            + [pltpu.SemaphoreType.DMA] * (2 * _NBUF)
        ),
    )
    return run(embed[:s])


# speedup vs baseline: 1.0399x; 1.0399x over previous
import functools
import math

import jax
import jax.numpy as jnp
from jax import lax
from jax.experimental import pallas as pl
from jax.experimental.pallas import tpu as pltpu
from jax.experimental.pallas import tpu_sc as plsc

_NC = 2
_NS = 16
_NW = _NC * _NS
_LANES = 16
_CHUNK_ROWS = 16
_NBUF = 7


def _sc_body(scale, n_chunks, d, in_hbm, out_hbm, *scratch):
    bufs = scratch[:_NBUF]
    isems = scratch[_NBUF:2 * _NBUF]
    osems = scratch[2 * _NBUF:3 * _NBUF]
    wid = lax.axis_index("s") * _NC + lax.axis_index("c")
    base = wid * (n_chunks * _CHUNK_ROWS)
    vecs_per_row = d // _LANES
    assert vecs_per_row & (vecs_per_row - 1) == 0
    row_shift = vecs_per_row.bit_length() - 1
    vecs = _CHUNK_ROWS * vecs_per_row

    def src(c):
        return in_hbm.at[pl.ds(base + c * _CHUNK_ROWS, _CHUNK_ROWS), :]

    def dst(c):
        return out_hbm.at[pl.ds(base + c * _CHUNK_ROWS, _CHUNK_ROWS), :]

    in_h = {}
    out_h = {}
    for j in range(min(_NBUF - 1, n_chunks)):
        in_h[j] = pltpu.async_copy(src(j), bufs[j % _NBUF], isems[j % _NBUF])
    for c in range(n_chunks):
        b = c % _NBUF
        nxt = c + _NBUF - 1
        if nxt < n_chunks:
            if c - 1 >= 0:
                out_h[c - 1].wait()
            in_h[nxt] = pltpu.async_copy(
                src(nxt), bufs[nxt % _NBUF], isems[nxt % _NBUF])
        in_h[c].wait()

        @plsc.parallel_loop(0, vecs, step=1, unroll=8)
        def _scale_one(i):
            r = lax.shift_right_logical(i, row_shift)
            col = pl.multiple_of(
                lax.shift_left(lax.bitwise_and(i, vecs_per_row - 1), 4),
                _LANES)
            sl = pl.ds(col, _LANES)
            bufs[b][r, sl] = bufs[b][r, sl] * scale

        out_h[c] = pltpu.async_copy(bufs[b], dst(c), osems[b])
    for c in range(max(0, n_chunks - _NBUF), n_chunks):
        out_h[c].wait()


def kernel(x, embed):
    s = x.shape[-2]
    d = embed.shape[-1]
    scale = 1.0 / math.sqrt(d)
    assert s % (_NW * _CHUNK_ROWS) == 0 and d % _LANES == 0
    n_chunks = s // (_NW * _CHUNK_ROWS)

    mesh = plsc.VectorSubcoreMesh(
        core_axis_name="c", subcore_axis_name="s",
        num_cores=_NC, num_subcores=_NS)
    run = pl.kernel(
        functools.partial(_sc_body, scale, n_chunks, d),
        out_type=jax.ShapeDtypeStruct((s, d), embed.dtype),
        mesh=mesh,
        scratch_types=(
            [pltpu.VMEM((_CHUNK_ROWS, d), embed.dtype) for _ in range(_NBUF)]
            + [pltpu.SemaphoreType.DMA] * (2 * _NBUF)
        ),
    )
    return run(embed[:s])
